# baseline (speedup 1.0000x reference)
import functools

import jax
import jax.numpy as jnp
from jax import lax
from jax.experimental import pallas as pl
from jax.experimental.pallas import tpu as pltpu
from jax.experimental.pallas import tpu_sc as plsc

_MODE = 0.02
_THRESHOLD = 0.1

_K = 4096
_N = 4096

_KSC = 1024
_KTC = _K - _KSC
_BK = 512

_NC, _NS = 2, 16
_NW = _NC * _NS
_RPW = _KSC // _NW
_SC_BK = 16
_G = 8


def _tc_body(x_ref, w_ref, b_ref, o_ref):
    i = pl.program_id(0)
    xm = x_ref[...] - _MODE
    v = jnp.where(jnp.abs(xm) > _THRESHOLD, xm, 0.0) + _MODE
    partial = jnp.sum(w_ref[...] * v, axis=0, keepdims=True)

    @pl.when(i == 0)
    def _():
        o_ref[...] = b_ref[...] + partial

    @pl.when(i > 0)
    def _():
        o_ref[...] += partial


def _tc_matvec(x_col, w, b2):
    return pl.pallas_call(
        _tc_body,
        grid=(_KTC // _BK,),
        in_specs=[
            pl.BlockSpec((_BK, 1), lambda i: (i, 0)),
            pl.BlockSpec((_BK, _N), lambda i: (i, 0)),
            pl.BlockSpec((1, _N), lambda i: (0, 0)),
        ],
        out_specs=pl.BlockSpec((1, _N), lambda i: (0, 0)),
        out_shape=jax.ShapeDtypeStruct((1, _N), jnp.float32),
    )(x_col, w, b2)


def _sc_body(w_hbm, x_hbm, o_hbm, wbuf, vbuf, acc):
    wid = lax.axis_index("s") * _NC + lax.axis_index("c")
    base = wid * _RPW

    pltpu.sync_copy(x_hbm.at[pl.ds(base, _RPW)], vbuf)

    @pl.loop(0, _RPW, step=16)
    def _(i):
        xv = vbuf[pl.ds(i, 16)]
        xm = xv - _MODE
        vbuf[pl.ds(i, 16)] = jnp.where(jnp.abs(xm) > _THRESHOLD, xm, 0.0) + _MODE

    @pl.loop(0, _N, step=16)
    def _(j):
        acc[pl.ds(j, 16)] = jnp.zeros((16,), jnp.float32)

    @pl.loop(0, _RPW, step=_SC_BK)
    def _(rb):
        pltpu.sync_copy(w_hbm.at[pl.ds(base + rb, _SC_BK)], wbuf)
        vv = vbuf[pl.ds(rb, 16)]

        @pl.loop(0, _N, step=16 * _G)
        def _(cg):
            accs = [acc[pl.ds(cg + 16 * g, 16)] for g in range(_G)]
            for r in range(_SC_BK):
                for g in range(_G):
                    accs[g] = accs[g] + vv[r] * wbuf[r, pl.ds(cg + 16 * g, 16)]
            for g in range(_G):
                acc[pl.ds(cg + 16 * g, 16)] = accs[g]

    pltpu.sync_copy(acc, o_hbm.at[wid])


@functools.partial(jax.jit, static_argnames=())
def _sc_partials(w_sc, x_sc):
    mesh = plsc.VectorSubcoreMesh(core_axis_name="c", subcore_axis_name="s")
    f = pl.kernel(
        _sc_body,
        mesh=mesh,
        out_type=jax.ShapeDtypeStruct((_NW, _N), jnp.float32),
        scratch_types=[
            pltpu.VMEM((_SC_BK, _N), jnp.float32),
            pltpu.VMEM((_RPW,), jnp.float32),
            pltpu.VMEM((_N,), jnp.float32),
        ],
    )
    return f(w_sc, x_sc)


def kernel(x, weight_t, bias):
    xf = x.reshape(_K)
    x_tc = xf[:_KTC].reshape(_KTC, 1)
    x_sc = xf[_KTC:]
    w_tc = weight_t[:_KTC]
    w_sc = weight_t[_KTC:]
    b2 = bias.reshape(1, _N)
    parts = _sc_partials(w_sc, x_sc)
    y_tc = _tc_matvec(x_tc, w_tc, b2)
    return y_tc + jnp.sum(parts, axis=0, keepdims=True)


# speedup vs baseline: 1.9825x; 1.9825x over previous
import functools

import jax
import jax.numpy as jnp
from jax import lax
from jax.experimental import pallas as pl
from jax.experimental.pallas import tpu as pltpu
from jax.experimental.pallas import tpu_sc as plsc

_MODE = 0.02
_THRESHOLD = 0.1

_K = 4096
_N = 4096

_KSC = 1536
_KTC = _K - _KSC
_BK = 512

_NC, _NS = 2, 16
_NW = _NC * _NS
_RPW = _KSC // _NW
_SC_BK = 8
_G = 8


def _tc_body(x_ref, w_ref, b_ref, o_ref):
    i = pl.program_id(0)
    xm = x_ref[...] - _MODE
    v = jnp.where(jnp.abs(xm) > _THRESHOLD, xm, 0.0) + _MODE
    partial = jnp.sum(w_ref[...] * v, axis=0, keepdims=True)

    @pl.when(i == 0)
    def _():
        o_ref[...] = b_ref[...] + partial

    @pl.when(i > 0)
    def _():
        o_ref[...] += partial


def _tc_matvec(x_col, w, b2):
    return pl.pallas_call(
        _tc_body,
        grid=(_KTC // _BK,),
        in_specs=[
            pl.BlockSpec((_BK, 1), lambda i: (i, 0)),
            pl.BlockSpec((_BK, _N), lambda i: (i, 0)),
            pl.BlockSpec((1, _N), lambda i: (0, 0)),
        ],
        out_specs=pl.BlockSpec((1, _N), lambda i: (0, 0)),
        out_shape=jax.ShapeDtypeStruct((1, _N), jnp.float32),
    )(x_col, w, b2)


def _sc_accum_block(acc, wbuf, vv, lane0):
    @pl.loop(0, _N, step=16 * _G)
    def _(cg):
        accs = [acc[pl.ds(cg + 16 * g, 16)] for g in range(_G)]
        for r in range(_SC_BK):
            for g in range(_G):
                accs[g] = accs[g] + vv[lane0 + r] * wbuf[r, pl.ds(cg + 16 * g, 16)]
        for g in range(_G):
            acc[pl.ds(cg + 16 * g, 16)] = accs[g]


def _sc_body(w_hbm, x_hbm, o_hbm, buf_a, buf_b, vbuf, acc, sem_a, sem_b):
    wid = lax.axis_index("s") * _NC + lax.axis_index("c")
    base = _KTC + wid * _RPW

    pltpu.sync_copy(x_hbm.at[pl.ds(base, _RPW)], vbuf)

    @pl.loop(0, _RPW, step=16)
    def _(i):
        xv = vbuf[pl.ds(i, 16)]
        xm = xv - _MODE
        vbuf[pl.ds(i, 16)] = jnp.where(jnp.abs(xm) > _THRESHOLD, xm, 0.0) + _MODE

    @pl.loop(0, _N, step=16)
    def _(j):
        acc[pl.ds(j, 16)] = jnp.zeros((16,), jnp.float32)

    pltpu.async_copy(w_hbm.at[pl.ds(base, _SC_BK)], buf_a, sem_a)

    @pl.loop(0, _RPW, step=2 * _SC_BK)
    def _(p):
        vv = vbuf[pl.ds(p, 16)]
        pltpu.async_copy(w_hbm.at[pl.ds(base + p + _SC_BK, _SC_BK)], buf_b, sem_b)
        pltpu.make_async_copy(w_hbm.at[pl.ds(base, _SC_BK)], buf_a, sem_a).wait()
        _sc_accum_block(acc, buf_a, vv, 0)

        @pl.when(p + 2 * _SC_BK < _RPW)
        def _():
            pltpu.async_copy(
                w_hbm.at[pl.ds(base + p + 2 * _SC_BK, _SC_BK)], buf_a, sem_a
            )

        pltpu.make_async_copy(w_hbm.at[pl.ds(base, _SC_BK)], buf_b, sem_b).wait()
        _sc_accum_block(acc, buf_b, vv, _SC_BK)

    pltpu.sync_copy(acc, o_hbm.at[wid])


def _sc_partials(w, xf):
    mesh = plsc.VectorSubcoreMesh(core_axis_name="c", subcore_axis_name="s")
    f = pl.kernel(
        _sc_body,
        mesh=mesh,
        out_type=jax.ShapeDtypeStruct((_NW, _N), jnp.float32),
        scratch_types=[
            pltpu.VMEM((_SC_BK, _N), jnp.float32),
            pltpu.VMEM((_SC_BK, _N), jnp.float32),
            pltpu.VMEM((_RPW,), jnp.float32),
            pltpu.VMEM((_N,), jnp.float32),
            pltpu.SemaphoreType.DMA,
            pltpu.SemaphoreType.DMA,
        ],
    )
    return f(w, xf)


def kernel(x, weight_t, bias):
    xf = x.reshape(_K)
    x_col = x.reshape(_K, 1)
    b2 = bias.reshape(1, _N)
    parts = _sc_partials(weight_t, xf)
    y_tc = _tc_matvec(x_col, weight_t, b2)
    return y_tc + jnp.sum(parts, axis=0, keepdims=True)


# speedup vs baseline: 3.4462x; 1.7383x over previous
import jax
import jax.numpy as jnp
from jax.experimental import pallas as pl

_MODE = 0.02
_THRESHOLD = 0.1

_BK = 512


def _matvec_body(x_ref, w_ref, b_ref, o_ref):
    i = pl.program_id(0)
    xb = x_ref[...]
    xm = xb - _MODE
    v = jnp.where(jnp.abs(xm) > _THRESHOLD, xm, 0.0) + _MODE
    partial = jnp.sum(w_ref[...] * v, axis=0, keepdims=True)

    @pl.when(i == 0)
    def _():
        o_ref[...] = b_ref[...] + partial

    @pl.when(i > 0)
    def _():
        o_ref[...] += partial


def kernel(x, weight_t, bias):
    k, n = weight_t.shape
    xa = x.reshape(k, 1)
    b2 = bias.reshape(1, n)
    out = pl.pallas_call(
        _matvec_body,
        grid=(k // _BK,),
        in_specs=[
            pl.BlockSpec((_BK, 1), lambda i: (i, 0)),
            pl.BlockSpec((_BK, n), lambda i: (i, 0)),
            pl.BlockSpec((1, n), lambda i: (0, 0)),
        ],
        out_specs=pl.BlockSpec((1, n), lambda i: (0, 0)),
        out_shape=jax.ShapeDtypeStruct((1, n), jnp.float32),
    )(xa, weight_t, b2)
    return out


# speedup vs baseline: 4.0053x; 1.1622x over previous
import jax
import jax.numpy as jnp
from jax.experimental import pallas as pl

_MODE = 0.02
_THRESHOLD = 0.1

_BK = 512


def _matvec_body(x_ref, w_ref, b_ref, o_ref):
    i = pl.program_id(0)
    xb = x_ref[...]
    xm = xb - _MODE
    v = jnp.where(jnp.abs(xm) > _THRESHOLD, xm, 0.0) + _MODE
    vc = v.reshape(_BK, 1)
    partial = jnp.sum(w_ref[...] * vc, axis=0, keepdims=True)

    @pl.when(i == 0)
    def _():
        o_ref[...] = b_ref[...] + partial

    @pl.when(i > 0)
    def _():
        o_ref[...] += partial


def kernel(x, weight_t, bias):
    k, n = weight_t.shape
    b2 = bias.reshape(1, n)
    out = pl.pallas_call(
        _matvec_body,
        grid=(k // _BK,),
        in_specs=[
            pl.BlockSpec((1, _BK), lambda i: (0, i)),
            pl.BlockSpec((_BK, n), lambda i: (i, 0)),
            pl.BlockSpec((1, n), lambda i: (0, 0)),
        ],
        out_specs=pl.BlockSpec((1, n), lambda i: (0, 0)),
        out_shape=jax.ShapeDtypeStruct((1, n), jnp.float32),
    )(x, weight_t, b2)
    return out
